# baseline (speedup 1.0000x reference)
import jax
import jax.numpy as jnp
from jax import lax
from jax.experimental import pallas as pl
from jax.experimental.pallas import tpu as pltpu
from jax.experimental.pallas import tpu_sc as plsc

_B = 4096
_S = 200
_K = 4
_ROWS_TABLE = 1000001
_NC = 2
_NS = 16
_NW = _NC * _NS
_ROWS_W = _B // _NW
_ELEMS_W = _ROWS_W * _S
_PAIRS = _ROWS_W // 2
_CHUNKS = (2 * _S) // 16
_L = 16


def _body(x_hbm, xe_hbm, w_hbm, out_hbm, x_v, xe_v, base_v, idx_v, out_v, sem):
  wid = lax.axis_index("s") * _NC + lax.axis_index("c")

  pltpu.sync_copy(x_hbm.at[pl.ds(wid * (_ROWS_W * _S * 2), _ROWS_W * _S * 2)],
                  x_v)
  pltpu.sync_copy(xe_hbm.at[pl.ds(wid * (_ROWS_W * 3), _ROWS_W * 3)], xe_v)

  iota = lax.iota(jnp.int32, _L)

  def base_step(r, carry):
    words = (r * _L + iota) * 3 + 2
    strat = plsc.load_gather(xe_v, [words])
    base_v[pl.ds(r * _L, _L)] = jnp.minimum(strat, _K - 1) * _ROWS_TABLE
    return carry

  lax.fori_loop(0, _ROWS_W // _L, base_step, 0)

  def pair_step(j, carry):
    r2 = 2 * j
    off = (2 * _S) * j
    for c in range(_CHUNKS):
      p = iota + (c * _L)
      row = r2 + (p >= _S).astype(jnp.int32)
      xi = plsc.load_gather(x_v, [2 * p + 2 * off])
      fb = plsc.load_gather(base_v, [row])
      idx_v[pl.ds(off + c * _L, _L)] = fb + xi
    return carry

  lax.fori_loop(0, _PAIRS, pair_step, 0)

  pltpu.async_copy(w_hbm.at[idx_v], out_v, sem).wait()
  pltpu.sync_copy(out_v, out_hbm.at[pl.ds(wid * _ELEMS_W, _ELEMS_W)])


@jax.jit
def _stratified_gather(x_flat, xe_flat, w_cat):
  mesh = plsc.VectorSubcoreMesh(
      core_axis_name="c", subcore_axis_name="s", num_cores=_NC,
      num_subcores=_NS)
  run = pl.kernel(
      _body,
      out_type=jax.ShapeDtypeStruct((_B * _S,), jnp.float32),
      mesh=mesh,
      compiler_params=pltpu.CompilerParams(needs_layout_passes=False),
      scratch_types=[
          pltpu.VMEM((_ROWS_W * _S * 2,), jnp.int32),
          pltpu.VMEM((_ROWS_W * 3,), jnp.int32),
          pltpu.VMEM((_ROWS_W,), jnp.int32),
          pltpu.VMEM((_ELEMS_W,), jnp.int32),
          pltpu.VMEM((_ELEMS_W,), jnp.float32),
          pltpu.SemaphoreType.DMA,
      ],
  )
  return run(x_flat, xe_flat, w_cat)


def kernel(x, x_extra, W0, W1, W2, W3):
  w_cat = jnp.concatenate(
      [W0.reshape(-1), W1.reshape(-1), W2.reshape(-1), W3.reshape(-1)])
  out = _stratified_gather(x.reshape(-1), x_extra.reshape(-1), w_cat)
  return out.reshape(_B, _S, 1)
